# baseline (speedup 1.0000x reference)
import functools

import jax
import jax.numpy as jnp
from jax import lax
from jax.experimental import pallas as pl
from jax.experimental.pallas import tpu as pltpu
from jax.experimental.pallas import tpu_sc as plsc

N = 10000
E = 320000
D = 128

NC = 2
NS = 16
NW = NC * NS
EPW = E // NW
K = 125
CH = EPW // K
GS = 16
NG = CH // GS
NB = 2
SLOT = 128
NP = 10240
RPS = NP // NS
ZR = 64
DEGW = 128


def _sc_mesh():
    return plsc.VectorSubcoreMesh(core_axis_name="c", subcore_axis_name="s")


def _zero_stripe(zbuf, sh, s, width):
    def _fz(i, _):
        for j in range(width // 16):
            zbuf[i, pl.ds(j * 16, 16)] = jnp.zeros((16,), jnp.float32)
        return 0

    lax.fori_loop(0, ZR, _fz, 0)
    for t in range(RPS // ZR):
        pltpu.sync_copy(zbuf, sh.at[pl.ds(s * RPS + t * ZR, ZR)])


def _deg_body(dst_hbm, out_hbm, dstb, onesb, zbuf, deg_sh, sem):
    c = lax.axis_index("c")
    s = lax.axis_index("s")

    def _fill(i, _):
        for j in range(DEGW // 16):
            onesb[i, pl.ds(j * 16, 16)] = jnp.ones((16,), jnp.float32)
        return 0

    lax.fori_loop(0, K, _fill, 0)
    _zero_stripe(zbuf, deg_sh, s, DEGW)

    w = c * NS + s
    pltpu.sync_copy(dst_hbm.at[pl.ds(w * CH, CH)], dstb)
    plsc.subcore_barrier()

    def _chunk(i, _):
        pltpu.sync_copy(onesb, deg_sh.at[dstb.at[i]], add=True)
        return 0

    lax.fori_loop(0, CH, _chunk, 0)
    plsc.subcore_barrier()
    pltpu.sync_copy(deg_sh.at[pl.ds(s * RPS, RPS)],
                    out_hbm.at[pl.ds(c * NP + s * RPS, RPS)])


def _deg_call(dst2):
    kfn = pl.kernel(
        _deg_body,
        out_type=jax.ShapeDtypeStruct((NC * NP, DEGW), jnp.float32),
        mesh=_sc_mesh(),
        scratch_types=[
            pltpu.VMEM((CH, K), jnp.int32),
            pltpu.VMEM((K, DEGW), jnp.float32),
            pltpu.VMEM((ZR, DEGW), jnp.float32),
            pltpu.VMEM_SHARED((NP, DEGW), jnp.float32),
            pltpu.SemaphoreType.DMA,
        ],
        name="gcn_deg_sc",
    )
    return kfn(dst2)


def _agg_body(h_hbm, src_hbm, dst_hbm, out_hbm, sb0, sb1, dstb, rows,
              agg_sh, ssem0, ssem1, *sems):
    c = lax.axis_index("c")
    s = lax.axis_index("s")
    w = c * NS + s

    sbufs = (sb0, sb1)
    ssems = (ssem0, ssem1)
    gsem = sems[:2 * NB]
    ssem = sems[2 * NB:]

    def _fz(i, _):
        for j in range(D // 16):
            rows[i, pl.ds(j * 16, 16)] = jnp.zeros((16,), jnp.float32)
        return 0

    lax.fori_loop(0, SLOT, _fz, 0)
    for t in range(RPS // SLOT):
        pltpu.sync_copy(rows.at[pl.ds(0, SLOT)],
                        agg_sh.at[pl.ds(s * RPS + t * SLOT, SLOT)])

    pltpu.sync_copy(dst_hbm.at[pl.ds(w * CH, CH)], dstb)
    pltpu.sync_copy(src_hbm.at[pl.ds(w * CH, GS)], sb0)
    pltpu.async_copy(src_hbm.at[pl.ds(w * CH + GS, GS)], sb1, ssem1)
    plsc.subcore_barrier()

    def _slot(b):
        return rows.at[pl.ds(b * SLOT, K)]

    KH = 64

    def _gather(j, b):
        blk, t = j // GS, j % GS
        sb = sbufs[blk % 2]
        g0 = h_hbm.at[sb.at[t, pl.ds(0, KH)]]
        g1 = h_hbm.at[sb.at[t, pl.ds(KH, K - KH)]]
        pltpu.async_copy(g0, rows.at[pl.ds(b * SLOT, KH)], gsem[b])
        pltpu.async_copy(g1, rows.at[pl.ds(b * SLOT + KH, K - KH)],
                         gsem[NB + b])
        return (g0, g1)

    def _gwait(b):
        g0, g1 = pending[b]
        pltpu.make_async_copy(g0, rows.at[pl.ds(b * SLOT, KH)],
                              gsem[b]).wait()
        pltpu.make_async_copy(g1, rows.at[pl.ds(b * SLOT + KH, K - KH)],
                              gsem[NB + b]).wait()

    pending = [None] * NB
    for b in range(NB):
        pending[b] = _gather(b, b)

    scat = [None] * NB
    for i in range(CH):
        b = i % NB
        _gwait(b)
        dref = agg_sh.at[dstb.at[i]]
        pltpu.async_copy(_slot(b), dref, ssem[b], add=True)
        scat[b] = dref
        j = i + NB
        if j < CH:
            blk, t = j // GS, j % GS
            if t == 0:
                pltpu.make_async_copy(
                    src_hbm.at[pl.ds(w * CH + blk * GS, GS)],
                    sbufs[blk % 2], ssems[blk % 2]).wait()
            if t == NB and blk + 1 < NG:
                pltpu.async_copy(
                    src_hbm.at[pl.ds(w * CH + (blk + 1) * GS, GS)],
                    sbufs[(blk + 1) % 2], ssems[(blk + 1) % 2])
            pltpu.make_async_copy(_slot(b), scat[b], ssem[b]).wait()
            pending[b] = _gather(j, b)

    for i in range(CH - NB, CH):
        b = i % NB
        pltpu.make_async_copy(_slot(b), scat[b], ssem[b]).wait()
    plsc.subcore_barrier()
    pltpu.sync_copy(agg_sh.at[pl.ds(s * RPS, RPS)],
                    out_hbm.at[pl.ds(c * NP + s * RPS, RPS)])


def _agg_call(h, src2, dst2):
    kfn = pl.kernel(
        _agg_body,
        out_type=jax.ShapeDtypeStruct((NC * NP, D), jnp.float32),
        mesh=_sc_mesh(),
        scratch_types=[
            pltpu.VMEM((GS, K), jnp.int32),
            pltpu.VMEM((GS, K), jnp.int32),
            pltpu.VMEM((CH, K), jnp.int32),
            pltpu.VMEM((NB * SLOT, D), jnp.float32),
            pltpu.VMEM_SHARED((NP, D), jnp.float32),
            pltpu.SemaphoreType.DMA,
            pltpu.SemaphoreType.DMA,
        ] + [pltpu.SemaphoreType.DMA] * (3 * NB),
        name="gcn_agg_sc",
    )
    return kfn(h, src2, dst2)


BM = 1000


def _mm_body(x_ref, w_ref, o_ref):
    o_ref[...] = jnp.dot(x_ref[...], w_ref[...],
                         preferred_element_type=jnp.float32)


def _matmul(x, W):
    return pl.pallas_call(
        _mm_body,
        grid=(N // BM,),
        in_specs=[pl.BlockSpec((BM, D), lambda i: (i, 0)),
                  pl.BlockSpec((D, D), lambda i: (0, 0))],
        out_specs=pl.BlockSpec((BM, D), lambda i: (i, 0)),
        out_shape=jax.ShapeDtypeStruct((N, D), jnp.float32),
    )(x, W)


def _deg_inv(da, db):
    deg = (da + db)[:, 0:1]
    return 1.0 / jnp.maximum(deg, 1.0)


def _fused_body(pa_ref, pb_ref, da_ref, db_ref, w_ref, o_ref):
    t = (pa_ref[...] + pb_ref[...]) * _deg_inv(da_ref[...], db_ref[...])
    t = jnp.maximum(t, 0.0)
    mu = jnp.mean(t, axis=-1, keepdims=True)
    var = jnp.mean((t - mu) ** 2, axis=-1, keepdims=True)
    t = (t - mu) * lax.rsqrt(var + 1e-9)
    o_ref[...] = jnp.dot(t, w_ref[...], preferred_element_type=jnp.float32)


def _fused(pa, pb, da, db, W):
    return pl.pallas_call(
        _fused_body,
        grid=(N // BM,),
        in_specs=[pl.BlockSpec((BM, D), lambda i: (i, 0)),
                  pl.BlockSpec((BM, D), lambda i: (i, 0)),
                  pl.BlockSpec((BM, DEGW), lambda i: (i, 0)),
                  pl.BlockSpec((BM, DEGW), lambda i: (i, 0)),
                  pl.BlockSpec((D, D), lambda i: (0, 0))],
        out_specs=pl.BlockSpec((BM, D), lambda i: (i, 0)),
        out_shape=jax.ShapeDtypeStruct((N, D), jnp.float32),
    )(pa, pb, da, db, W)


def _final_body(pa_ref, pb_ref, da_ref, db_ref, o_ref):
    o_ref[...] = (pa_ref[...] + pb_ref[...]) * _deg_inv(da_ref[...],
                                                        db_ref[...])


def _final(pa, pb, da, db):
    return pl.pallas_call(
        _final_body,
        grid=(N // BM,),
        in_specs=[pl.BlockSpec((BM, D), lambda i: (i, 0)),
                  pl.BlockSpec((BM, D), lambda i: (i, 0)),
                  pl.BlockSpec((BM, DEGW), lambda i: (i, 0)),
                  pl.BlockSpec((BM, DEGW), lambda i: (i, 0))],
        out_specs=pl.BlockSpec((BM, D), lambda i: (i, 0)),
        out_shape=jax.ShapeDtypeStruct((N, D), jnp.float32),
    )(pa, pb, da, db)


def kernel(sparse_adj, feats, W0, W1, W2):
    src2 = sparse_adj[0].reshape(E // K, K)
    dst2 = sparse_adj[1].reshape(E // K, K)

    degp = _deg_call(dst2)
    da, db = degp[:N], degp[NP:NP + N]

    h = _matmul(feats, W0)
    p = _agg_call(h, src2, dst2)
    h = _fused(p[:N], p[NP:NP + N], da, db, W1)
    p = _agg_call(h, src2, dst2)
    h = _fused(p[:N], p[NP:NP + N], da, db, W2)
    p = _agg_call(h, src2, dst2)
    return _final(p[:N], p[NP:NP + N], da, db)


# speedup vs baseline: 1.0230x; 1.0230x over previous
import functools

import jax
import jax.numpy as jnp
from jax import lax
from jax.experimental import pallas as pl
from jax.experimental.pallas import tpu as pltpu
from jax.experimental.pallas import tpu_sc as plsc

N = 10000
E = 320000
D = 128

NC = 2
NS = 16
NW = NC * NS
EPW = E // NW
K = 125
CH = EPW // K
GS = 16
NG = CH // GS
NB = 2
SLOT = 128
NP = 10240
RPS = NP // NS
ZR = 64
DEGW = 128


def _sc_mesh():
    return plsc.VectorSubcoreMesh(core_axis_name="c", subcore_axis_name="s")


def _zero_stripe(zbuf, sh, s, width):
    def _fz(i, _):
        for j in range(width // 16):
            zbuf[i, pl.ds(j * 16, 16)] = jnp.zeros((16,), jnp.float32)
        return 0

    lax.fori_loop(0, ZR, _fz, 0)
    for t in range(RPS // ZR):
        pltpu.sync_copy(zbuf, sh.at[pl.ds(s * RPS + t * ZR, ZR)])


def _deg_body(dst_hbm, out_hbm, dstb, onesb, zbuf, deg_sh, sem):
    c = lax.axis_index("c")
    s = lax.axis_index("s")

    def _fill(i, _):
        for j in range(DEGW // 16):
            onesb[i, pl.ds(j * 16, 16)] = jnp.ones((16,), jnp.float32)
        return 0

    lax.fori_loop(0, K, _fill, 0)
    _zero_stripe(zbuf, deg_sh, s, DEGW)

    w = c * NS + s
    pltpu.sync_copy(dst_hbm.at[pl.ds(w * CH, CH)], dstb)
    plsc.subcore_barrier()

    def _chunk(i, _):
        pltpu.sync_copy(onesb, deg_sh.at[dstb.at[i]], add=True)
        return 0

    lax.fori_loop(0, CH, _chunk, 0)
    plsc.subcore_barrier()
    pltpu.sync_copy(deg_sh.at[pl.ds(s * RPS, RPS)],
                    out_hbm.at[pl.ds(c * NP + s * RPS, RPS)])


def _deg_call(dst2):
    kfn = pl.kernel(
        _deg_body,
        out_type=jax.ShapeDtypeStruct((NC * NP, DEGW), jnp.float32),
        mesh=_sc_mesh(),
        scratch_types=[
            pltpu.VMEM((CH, K), jnp.int32),
            pltpu.VMEM((K, DEGW), jnp.float32),
            pltpu.VMEM((ZR, DEGW), jnp.float32),
            pltpu.VMEM_SHARED((NP, DEGW), jnp.float32),
            pltpu.SemaphoreType.DMA,
        ],
        name="gcn_deg_sc",
    )
    return kfn(dst2)


def _agg_body(h_hbm, src_hbm, dst_hbm, out_hbm, sb0, sb1, dstb, rows,
              agg_sh, ssem0, ssem1, *sems):
    c = lax.axis_index("c")
    s = lax.axis_index("s")
    w = c * NS + s

    sbufs = (sb0, sb1)
    ssems = (ssem0, ssem1)
    gsem = sems[:NB]
    ssem = sems[NB:]

    def _fz(i, _):
        for j in range(D // 16):
            rows[i, pl.ds(j * 16, 16)] = jnp.zeros((16,), jnp.float32)
        return 0

    lax.fori_loop(0, SLOT, _fz, 0)
    for t in range(RPS // SLOT):
        pltpu.sync_copy(rows.at[pl.ds(0, SLOT)],
                        agg_sh.at[pl.ds(s * RPS + t * SLOT, SLOT)])

    pltpu.sync_copy(dst_hbm.at[pl.ds(w * CH, CH)], dstb)
    pltpu.sync_copy(src_hbm.at[pl.ds(w * CH, GS)], sb0)
    pltpu.async_copy(src_hbm.at[pl.ds(w * CH + GS, GS)], sb1, ssem1)
    plsc.subcore_barrier()

    def _slot(b):
        return rows.at[pl.ds(b * SLOT, K)]

    def _gather(j, b):
        blk, t = j // GS, j % GS
        gref = h_hbm.at[sbufs[blk % 2].at[t]]
        pltpu.async_copy(gref, _slot(b), gsem[b])
        return gref

    def _gwait(b):
        pltpu.make_async_copy(pending[b], _slot(b), gsem[b]).wait()

    pending = [None] * NB
    for b in range(NB):
        pending[b] = _gather(b, b)

    scat = [None] * NB
    for i in range(CH):
        b = i % NB
        _gwait(b)
        dref = agg_sh.at[dstb.at[i]]
        pltpu.async_copy(_slot(b), dref, ssem[b], add=True)
        scat[b] = dref
        j = i + NB
        if j < CH:
            blk, t = j // GS, j % GS
            if t == 0:
                pltpu.make_async_copy(
                    src_hbm.at[pl.ds(w * CH + blk * GS, GS)],
                    sbufs[blk % 2], ssems[blk % 2]).wait()
            if t == NB and blk + 1 < NG:
                pltpu.async_copy(
                    src_hbm.at[pl.ds(w * CH + (blk + 1) * GS, GS)],
                    sbufs[(blk + 1) % 2], ssems[(blk + 1) % 2])
            pltpu.make_async_copy(_slot(b), scat[b], ssem[b]).wait()
            pending[b] = _gather(j, b)

    for i in range(CH - NB, CH):
        b = i % NB
        pltpu.make_async_copy(_slot(b), scat[b], ssem[b]).wait()
    plsc.subcore_barrier()
    pltpu.sync_copy(agg_sh.at[pl.ds(s * RPS, RPS)],
                    out_hbm.at[pl.ds(c * NP + s * RPS, RPS)])


def _agg_call(h, src2, dst2):
    kfn = pl.kernel(
        _agg_body,
        out_type=jax.ShapeDtypeStruct((NC * NP, D), jnp.float32),
        mesh=_sc_mesh(),
        scratch_types=[
            pltpu.VMEM((GS, K), jnp.int32),
            pltpu.VMEM((GS, K), jnp.int32),
            pltpu.VMEM((CH, K), jnp.int32),
            pltpu.VMEM((NB * SLOT, D), jnp.float32),
            pltpu.VMEM_SHARED((NP, D), jnp.float32),
            pltpu.SemaphoreType.DMA,
            pltpu.SemaphoreType.DMA,
        ] + [pltpu.SemaphoreType.DMA] * (2 * NB),
        name="gcn_agg_sc",
    )
    return kfn(h, src2, dst2)


BM = 1000


def _mm_body(x_ref, w_ref, o_ref):
    o_ref[...] = jnp.dot(x_ref[...], w_ref[...],
                         preferred_element_type=jnp.float32)


def _matmul(x, W):
    return pl.pallas_call(
        _mm_body,
        grid=(N // BM,),
        in_specs=[pl.BlockSpec((BM, D), lambda i: (i, 0)),
                  pl.BlockSpec((D, D), lambda i: (0, 0))],
        out_specs=pl.BlockSpec((BM, D), lambda i: (i, 0)),
        out_shape=jax.ShapeDtypeStruct((N, D), jnp.float32),
    )(x, W)


def _deg_inv(da, db):
    deg = (da + db)[:, 0:1]
    return 1.0 / jnp.maximum(deg, 1.0)


def _fused_body(pa_ref, pb_ref, da_ref, db_ref, w_ref, o_ref):
    t = (pa_ref[...] + pb_ref[...]) * _deg_inv(da_ref[...], db_ref[...])
    t = jnp.maximum(t, 0.0)
    mu = jnp.mean(t, axis=-1, keepdims=True)
    var = jnp.mean((t - mu) ** 2, axis=-1, keepdims=True)
    t = (t - mu) * lax.rsqrt(var + 1e-9)
    o_ref[...] = jnp.dot(t, w_ref[...], preferred_element_type=jnp.float32)


def _fused(pa, pb, da, db, W):
    return pl.pallas_call(
        _fused_body,
        grid=(N // BM,),
        in_specs=[pl.BlockSpec((BM, D), lambda i: (i, 0)),
                  pl.BlockSpec((BM, D), lambda i: (i, 0)),
                  pl.BlockSpec((BM, DEGW), lambda i: (i, 0)),
                  pl.BlockSpec((BM, DEGW), lambda i: (i, 0)),
                  pl.BlockSpec((D, D), lambda i: (0, 0))],
        out_specs=pl.BlockSpec((BM, D), lambda i: (i, 0)),
        out_shape=jax.ShapeDtypeStruct((N, D), jnp.float32),
    )(pa, pb, da, db, W)


def _final_body(pa_ref, pb_ref, da_ref, db_ref, o_ref):
    o_ref[...] = (pa_ref[...] + pb_ref[...]) * _deg_inv(da_ref[...],
                                                        db_ref[...])


def _final(pa, pb, da, db):
    return pl.pallas_call(
        _final_body,
        grid=(N // BM,),
        in_specs=[pl.BlockSpec((BM, D), lambda i: (i, 0)),
                  pl.BlockSpec((BM, D), lambda i: (i, 0)),
                  pl.BlockSpec((BM, DEGW), lambda i: (i, 0)),
                  pl.BlockSpec((BM, DEGW), lambda i: (i, 0))],
        out_specs=pl.BlockSpec((BM, D), lambda i: (i, 0)),
        out_shape=jax.ShapeDtypeStruct((N, D), jnp.float32),
    )(pa, pb, da, db)


def kernel(sparse_adj, feats, W0, W1, W2):
    src2 = sparse_adj[0].reshape(E // K, K)
    dst2 = sparse_adj[1].reshape(E // K, K)

    degp = _deg_call(dst2)
    da, db = degp[:N], degp[NP:NP + N]

    h = _matmul(feats, W0)
    p = _agg_call(h, src2, dst2)
    h = _fused(p[:N], p[NP:NP + N], da, db, W1)
    p = _agg_call(h, src2, dst2)
    h = _fused(p[:N], p[NP:NP + N], da, db, W2)
    p = _agg_call(h, src2, dst2)
    return _final(p[:N], p[NP:NP + N], da, db)
